# baseline (speedup 1.0000x reference)
import functools

import jax
import jax.numpy as jnp
from jax import lax
from jax.experimental import pallas as pl
from jax.experimental.pallas import tpu as pltpu
from jax.experimental.pallas import tpu_sc as plsc

EPS = 1e-15
MAX_TANH = 15.0

NC = 2
NS = 16
NW = NC * NS


def _artanh(x):
    x = jnp.clip(x, -1 + 1e-7, 1 - 1e-7)
    return 0.5 * (jnp.log1p(x) - jnp.log1p(-x))


def _safe_norm(x):
    return jnp.sqrt(jnp.clip(jnp.sum(x * x, axis=-1, keepdims=True), EPS, None))


def _proj(x):
    norm = _safe_norm(x)
    maxnorm = 1.0 - 1e-3
    return jnp.where(norm > maxnorm, x / norm * maxnorm, x)


def _expmap0(u):
    u_norm = _safe_norm(u)
    return jnp.tanh(jnp.clip(u_norm, -MAX_TANH, MAX_TANH)) * u / u_norm


def _logmap0(x):
    x_norm = _safe_norm(x)
    return _artanh(x_norm) * x / x_norm


def _head_body(x_ref, wt_ref, b_ref, o_ref):
    x = x_ref[...]
    x_hyp = _proj(x)
    x_norm = _safe_norm(x_hyp)
    mx = jnp.dot(x_hyp, wt_ref[...], preferred_element_type=jnp.float32)
    mx_norm = _safe_norm(mx)
    mv = jnp.tanh(jnp.clip(mx_norm / x_norm * _artanh(x_norm),
                           -MAX_TANH, MAX_TANH)) * mx / mx_norm
    mv = _proj(mv)
    bias = _proj(_expmap0(b_ref[...]))
    x2 = jnp.sum(mv * mv, axis=-1, keepdims=True)
    y2 = jnp.sum(bias * bias, axis=-1, keepdims=True)
    xy = jnp.sum(mv * bias, axis=-1, keepdims=True)
    num = (1 + 2 * xy + y2) * mv + (1 - x2) * bias
    den = 1 + 2 * xy + x2 * y2
    h = _proj(num / jnp.clip(den, EPS, None))
    o_ref[...] = _logmap0(h)


def _tail_body(a_ref, b_ref, o_ref):
    agg = a_ref[...] + b_ref[...]
    h_agg = _proj(_expmap0(agg))
    out = _proj(_expmap0(jax.nn.relu(_logmap0(h_agg))))
    o_ref[...] = _logmap0(out)


def _sc_aggregate(x_tan, idx, zeros, *, n, d, acc_rows, chunks, k):
    rpt = acc_rows // NS
    mesh = plsc.VectorSubcoreMesh(core_axis_name="c", subcore_axis_name="s")

    @functools.partial(
        pl.kernel,
        mesh=mesh,
        out_type=jax.ShapeDtypeStruct((NC, acc_rows, d), jnp.float32),
        scratch_types=[
            pltpu.VMEM_SHARED((acc_rows, d), jnp.float32),
            pltpu.VMEM((2, k), jnp.int32),
            pltpu.VMEM((2, k), jnp.int32),
            pltpu.VMEM((k, d), jnp.float32),
            pltpu.VMEM((k, d), jnp.float32),
            pltpu.SemaphoreType.DMA,
            pltpu.SemaphoreType.DMA,
            pltpu.SemaphoreType.DMA,
            pltpu.SemaphoreType.DMA,
        ],
    )
    def agg_kernel(xtan_hbm, idx_hbm, zeros_hbm, out_hbm,
                   acc, idx0, idx1, rows0, rows1,
                   isem0, isem1, gsem0, gsem1):
        idxs = (idx0, idx1)
        rows = (rows0, rows1)
        isems = (isem0, isem1)
        gsems = (gsem0, gsem1)
        c = lax.axis_index("c")
        s = lax.axis_index("s")
        wid = s * NC + c
        z0 = pl.multiple_of(s * rpt, 8)
        pltpu.sync_copy(zeros_hbm.at[pl.ds(z0, rpt)],
                        acc.at[pl.ds(z0, rpt)])
        pltpu.sync_copy(idx_hbm.at[wid, 0], idx0)
        pltpu.async_copy(idx_hbm.at[wid, 1], idx1, isem1)
        plsc.subcore_barrier()
        pltpu.async_copy(xtan_hbm.at[idx0.at[0]], rows0, gsem0)

        def outer(t, carry):
            for b in range(2):
                g = t * 2 + b
                nb = 1 - b

                @pl.when(g + 1 < chunks)
                def _launch_next(b=b, nb=nb, g=g):
                    pltpu.make_async_copy(
                        idx_hbm.at[wid, 0], idxs[nb], isems[nb]).wait()
                    pltpu.async_copy(
                        xtan_hbm.at[idxs[nb].at[0]], rows[nb], gsems[nb])

                pltpu.make_async_copy(
                    xtan_hbm.at[idxs[b].at[0]], rows[b], gsems[b]).wait()
                pltpu.sync_copy(rows[b], acc.at[idxs[b].at[1]], add=True)

                @pl.when(g + 2 < chunks)
                def _prefetch_idx(b=b, g=g):
                    pltpu.async_copy(
                        idx_hbm.at[wid, g + 2], idxs[b], isems[b])
            return carry

        lax.fori_loop(0, chunks // 2, outer, 0)
        plsc.subcore_barrier()
        pltpu.sync_copy(acc.at[pl.ds(z0, rpt)],
                        out_hbm.at[c, pl.ds(z0, rpt)])

    return agg_kernel(x_tan, idx, zeros)


def kernel(x, adj, W, b):
    n, d = x.shape
    e = adj.shape[1]
    assert n % NS == 0 and d % 128 == 0

    rb = 400
    grid_h = n // rb
    x_tan = pl.pallas_call(
        _head_body,
        grid=(grid_h,),
        in_specs=[
            pl.BlockSpec((rb, d), lambda i: (i, 0)),
            pl.BlockSpec((d, d), lambda i: (0, 0)),
            pl.BlockSpec((1, d), lambda i: (0, 0)),
        ],
        out_specs=pl.BlockSpec((rb, d), lambda i: (i, 0)),
        out_shape=jax.ShapeDtypeStruct((n, d), jnp.float32),
    )(x, W.T, b.reshape(1, d))

    k = 128
    grain = NW * k * 2
    e_pad = ((e + grain - 1) // grain) * grain
    chunks = e_pad // (NW * k)
    src = adj[0].astype(jnp.int32)
    dst = adj[1].astype(jnp.int32)
    if e_pad != e:
        src = jnp.pad(src, (0, e_pad - e))
        dst = jnp.pad(dst, (0, e_pad - e), constant_values=n)
    idx = jnp.stack([src.reshape(NW, chunks, k),
                     dst.reshape(NW, chunks, k)], axis=2)
    acc_rows = ((n + 1 + NS * 8 - 1) // (NS * 8)) * (NS * 8)
    zeros = jnp.zeros((acc_rows, d), jnp.float32)
    partials = _sc_aggregate(x_tan, idx, zeros,
                             n=n, d=d, acc_rows=acc_rows,
                             chunks=chunks, k=k)

    out = pl.pallas_call(
        _tail_body,
        grid=(grid_h,),
        in_specs=[
            pl.BlockSpec((rb, d), lambda i: (i, 0)),
            pl.BlockSpec((rb, d), lambda i: (i, 0)),
        ],
        out_specs=pl.BlockSpec((rb, d), lambda i: (i, 0)),
        out_shape=jax.ShapeDtypeStruct((n, d), jnp.float32),
    )(partials[0], partials[1])
    return out


# speedup vs baseline: 2.5394x; 2.5394x over previous
import functools

import jax
import jax.numpy as jnp
from jax import lax
from jax.experimental import pallas as pl
from jax.experimental.pallas import tpu as pltpu
from jax.experimental.pallas import tpu_sc as plsc

EPS = 1e-15
MAX_TANH = 15.0

NC = 2
NS = 16
NW = NC * NS


def _artanh(x):
    x = jnp.clip(x, -1 + 1e-7, 1 - 1e-7)
    return 0.5 * (jnp.log1p(x) - jnp.log1p(-x))


def _safe_norm(x):
    return jnp.sqrt(jnp.clip(jnp.sum(x * x, axis=-1, keepdims=True), EPS, None))


def _proj(x):
    norm = _safe_norm(x)
    maxnorm = 1.0 - 1e-3
    return jnp.where(norm > maxnorm, x / norm * maxnorm, x)


def _expmap0(u):
    u_norm = _safe_norm(u)
    return jnp.tanh(jnp.clip(u_norm, -MAX_TANH, MAX_TANH)) * u / u_norm


def _logmap0(x):
    x_norm = _safe_norm(x)
    return _artanh(x_norm) * x / x_norm


def _head_body(x_ref, wt_ref, b_ref, o_ref):
    x = x_ref[...]
    x_hyp = _proj(x)
    x_norm = _safe_norm(x_hyp)
    mx = jnp.dot(x_hyp, wt_ref[...], preferred_element_type=jnp.float32)
    mx_norm = _safe_norm(mx)
    mv = jnp.tanh(jnp.clip(mx_norm / x_norm * _artanh(x_norm),
                           -MAX_TANH, MAX_TANH)) * mx / mx_norm
    mv = _proj(mv)
    bias = _proj(_expmap0(b_ref[...]))
    x2 = jnp.sum(mv * mv, axis=-1, keepdims=True)
    y2 = jnp.sum(bias * bias, axis=-1, keepdims=True)
    xy = jnp.sum(mv * bias, axis=-1, keepdims=True)
    num = (1 + 2 * xy + y2) * mv + (1 - x2) * bias
    den = 1 + 2 * xy + x2 * y2
    h = _proj(num / jnp.clip(den, EPS, None))
    o_ref[...] = _logmap0(h)


def _tail_body(a_ref, b_ref, o_ref):
    agg = a_ref[...] + b_ref[...]
    h_agg = _proj(_expmap0(agg))
    out = _proj(_expmap0(jax.nn.relu(_logmap0(h_agg))))
    o_ref[...] = _logmap0(out)


NR = 3


def _sc_aggregate(x_tan, packed, zeros, *, n, d, acc_rows, chunks, k):
    rpt = acc_rows // NS
    mesh = plsc.VectorSubcoreMesh(core_axis_name="c", subcore_axis_name="s")

    @functools.partial(
        pl.kernel,
        mesh=mesh,
        out_type=jax.ShapeDtypeStruct((NC, acc_rows, d), jnp.float32),
        scratch_types=[
            pltpu.VMEM_SHARED((acc_rows, d), jnp.float32),
            pltpu.VMEM((chunks, k), jnp.int32),
        ]
        + [pltpu.VMEM((2, k), jnp.int32) for _ in range(NR)]
        + [pltpu.VMEM((k, d), jnp.float32) for _ in range(NR)]
        + [pltpu.SemaphoreType.DMA for _ in range(2 * NR)],
    )
    def agg_kernel(xtan_hbm, packed_hbm, zeros_hbm, out_hbm,
                   acc, slab, *rest):
        idxb = rest[:NR]
        rows = rest[NR:2 * NR]
        gsems = rest[2 * NR:3 * NR]
        ssems = rest[3 * NR:]
        c = lax.axis_index("c")
        s = lax.axis_index("s")
        wid = s * NC + c

        def unpack(g, j):
            for i in range(k // 16):
                v = slab.at[g][pl.ds(i * 16, 16)]
                idxb[j].at[0][pl.ds(i * 16, 16)] = v & 0x3FFF
                idxb[j].at[1][pl.ds(i * 16, 16)] = v >> 14

        def start_gather(g, j):
            pltpu.async_copy(xtan_hbm.at[idxb[j].at[0]], rows[j], gsems[j])

        def wait_gather(j):
            pltpu.make_async_copy(
                xtan_hbm.at[idxb[j].at[0]], rows[j], gsems[j]).wait()

        def start_scatter(j):
            pltpu.async_copy(rows[j], acc.at[idxb[j].at[1]], ssems[j],
                             add=True)

        def wait_scatter(j):
            pltpu.make_async_copy(
                rows[j], acc.at[idxb[j].at[1]], ssems[j]).wait()

        z0 = pl.multiple_of(s * rpt, 8)
        pltpu.sync_copy(zeros_hbm.at[pl.ds(z0, rpt)],
                        acc.at[pl.ds(z0, rpt)])
        pltpu.sync_copy(packed_hbm.at[wid], slab)
        for j in range(NR):
            unpack(j, j)
            start_gather(j, j)
        plsc.subcore_barrier()

        def outer(t, carry):
            for j in range(NR):
                wait_gather(j)
                start_scatter(j)
            for j in range(NR):
                gf = t * NR + j + NR

                @pl.when(gf < chunks)
                def _refill(j=j, gf=gf):
                    wait_scatter(j)
                    unpack(gf, j)
                    start_gather(gf, j)
            return carry

        lax.fori_loop(0, chunks // NR, outer, 0)
        for j in range(NR):
            wait_scatter(j)
        plsc.subcore_barrier()
        pltpu.sync_copy(acc.at[pl.ds(z0, rpt)],
                        out_hbm.at[c, pl.ds(z0, rpt)])

    return agg_kernel(x_tan, packed, zeros)


def kernel(x, adj, W, b):
    n, d = x.shape
    e = adj.shape[1]
    assert n % NS == 0 and d % 128 == 0

    rb = 400
    grid_h = n // rb
    x_tan = pl.pallas_call(
        _head_body,
        grid=(grid_h,),
        in_specs=[
            pl.BlockSpec((rb, d), lambda i: (i, 0)),
            pl.BlockSpec((d, d), lambda i: (0, 0)),
            pl.BlockSpec((1, d), lambda i: (0, 0)),
        ],
        out_specs=pl.BlockSpec((rb, d), lambda i: (i, 0)),
        out_shape=jax.ShapeDtypeStruct((n, d), jnp.float32),
    )(x, W.T, b.reshape(1, d))

    assert n < (1 << 14)
    k = 80
    grain = NW * k * NR
    e_pad = ((e + grain - 1) // grain) * grain
    chunks = e_pad // (NW * k)
    acc_rows = ((n + 1 + NS * 8 - 1) // (NS * 8)) * (NS * 8)
    src = adj[0].astype(jnp.int32)
    dst = adj[1].astype(jnp.int32)
    if e_pad != e:
        fill = jnp.arange(e_pad - e, dtype=jnp.int32)
        src = jnp.concatenate([src, fill % n])
        dst = jnp.concatenate([dst, n + fill % (acc_rows - n)])
    packed = (dst * (1 << 14) + src).reshape(NW, chunks, k)
    zeros = jnp.zeros((acc_rows, d), jnp.float32)
    partials = _sc_aggregate(x_tan, packed, zeros,
                             n=n, d=d, acc_rows=acc_rows,
                             chunks=chunks, k=k)

    out = pl.pallas_call(
        _tail_body,
        grid=(grid_h,),
        in_specs=[
            pl.BlockSpec((rb, d), lambda i: (i, 0)),
            pl.BlockSpec((rb, d), lambda i: (i, 0)),
        ],
        out_specs=pl.BlockSpec((rb, d), lambda i: (i, 0)),
        out_shape=jax.ShapeDtypeStruct((n, d), jnp.float32),
    )(partials[0], partials[1])
    return out


# speedup vs baseline: 2.6369x; 1.0384x over previous
import functools

import jax
import jax.numpy as jnp
from jax import lax
from jax.experimental import pallas as pl
from jax.experimental.pallas import tpu as pltpu
from jax.experimental.pallas import tpu_sc as plsc

EPS = 1e-15
MAX_TANH = 15.0

NC = 2
NS = 16
NW = NC * NS


def _artanh(x):
    x = jnp.clip(x, -1 + 1e-7, 1 - 1e-7)
    return 0.5 * (jnp.log1p(x) - jnp.log1p(-x))


def _safe_norm(x):
    return jnp.sqrt(jnp.clip(jnp.sum(x * x, axis=-1, keepdims=True), EPS, None))


def _proj(x):
    norm = _safe_norm(x)
    maxnorm = 1.0 - 1e-3
    return jnp.where(norm > maxnorm, x / norm * maxnorm, x)


def _expmap0(u):
    u_norm = _safe_norm(u)
    return jnp.tanh(jnp.clip(u_norm, -MAX_TANH, MAX_TANH)) * u / u_norm


def _logmap0(x):
    x_norm = _safe_norm(x)
    return _artanh(x_norm) * x / x_norm


def _head_body(x_ref, wt_ref, b_ref, o_ref):
    x = x_ref[...]
    x_hyp = _proj(x)
    x_norm = _safe_norm(x_hyp)
    mx = jnp.dot(x_hyp, wt_ref[...], preferred_element_type=jnp.float32)
    mx_norm = _safe_norm(mx)
    mv = jnp.tanh(jnp.clip(mx_norm / x_norm * _artanh(x_norm),
                           -MAX_TANH, MAX_TANH)) * mx / mx_norm
    mv = _proj(mv)
    bias = _proj(_expmap0(b_ref[...]))
    x2 = jnp.sum(mv * mv, axis=-1, keepdims=True)
    y2 = jnp.sum(bias * bias, axis=-1, keepdims=True)
    xy = jnp.sum(mv * bias, axis=-1, keepdims=True)
    num = (1 + 2 * xy + y2) * mv + (1 - x2) * bias
    den = 1 + 2 * xy + x2 * y2
    h = _proj(num / jnp.clip(den, EPS, None))
    o_ref[...] = _logmap0(h)


def _tail_body(a_ref, b_ref, o_ref):
    agg = a_ref[...] + b_ref[...]
    h_agg = _proj(_expmap0(agg))
    out = _proj(_expmap0(jax.nn.relu(_logmap0(h_agg))))
    o_ref[...] = _logmap0(out)


NR = 4


def _sc_aggregate(x_tan, packed, zeros, *, n, d, acc_rows, chunks, k):
    rpt = acc_rows // NS
    mesh = plsc.VectorSubcoreMesh(core_axis_name="c", subcore_axis_name="s")

    @functools.partial(
        pl.kernel,
        mesh=mesh,
        out_type=jax.ShapeDtypeStruct((NC, acc_rows, d), jnp.float32),
        scratch_types=[
            pltpu.VMEM_SHARED((acc_rows, d), jnp.float32),
            pltpu.VMEM((chunks * k,), jnp.int32),
        ]
        + [pltpu.VMEM((2, k), jnp.int32) for _ in range(NR)]
        + [pltpu.VMEM((k, d), jnp.float32) for _ in range(NR)]
        + [pltpu.SemaphoreType.DMA for _ in range(2 * NR)],
    )
    def agg_kernel(xtan_hbm, packed_hbm, zeros_hbm, out_hbm,
                   acc, slab, *rest):
        idxb = rest[:NR]
        rows = rest[NR:2 * NR]
        gsems = rest[2 * NR:3 * NR]
        ssems = rest[3 * NR:]
        c = lax.axis_index("c")
        s = lax.axis_index("s")
        wid = s * NC + c

        def unpack(g, j):
            for i in range(k // 16):
                v = slab[pl.ds(g * k + i * 16, 16)]
                idxb[j].at[0][pl.ds(i * 16, 16)] = v & 0x3FFF
                idxb[j].at[1][pl.ds(i * 16, 16)] = v >> 14

        def start_gather(g, j):
            pltpu.async_copy(xtan_hbm.at[idxb[j].at[0]], rows[j], gsems[j])

        def wait_gather(j):
            pltpu.make_async_copy(
                xtan_hbm.at[idxb[j].at[0]], rows[j], gsems[j]).wait()

        def start_scatter(j):
            pltpu.async_copy(rows[j], acc.at[idxb[j].at[1]], ssems[j],
                             add=True)

        def wait_scatter(j):
            pltpu.make_async_copy(
                rows[j], acc.at[idxb[j].at[1]], ssems[j]).wait()

        z0 = pl.multiple_of(s * rpt, 8)
        pltpu.sync_copy(zeros_hbm.at[pl.ds(z0, rpt)],
                        acc.at[pl.ds(z0, rpt)])
        pltpu.sync_copy(packed_hbm.at[wid], slab)
        for j in range(NR):
            unpack(j, j)
            start_gather(j, j)
        plsc.subcore_barrier()

        def outer(t, carry):
            for j in range(NR):
                wait_gather(j)
                start_scatter(j)
            for j in range(NR):
                gf = t * NR + j + NR

                @pl.when(gf < chunks)
                def _refill(j=j, gf=gf):
                    wait_scatter(j)
                    unpack(gf, j)
                    start_gather(gf, j)
            return carry

        lax.fori_loop(0, chunks // NR, outer, 0)
        for j in range(NR):
            wait_scatter(j)
        plsc.subcore_barrier()
        pltpu.sync_copy(acc.at[pl.ds(z0, rpt)],
                        out_hbm.at[c, pl.ds(z0, rpt)])

    return agg_kernel(x_tan, packed, zeros)


def kernel(x, adj, W, b):
    n, d = x.shape
    e = adj.shape[1]
    assert n % NS == 0 and d % 128 == 0

    rb = 400
    grid_h = n // rb
    x_tan = pl.pallas_call(
        _head_body,
        grid=(grid_h,),
        in_specs=[
            pl.BlockSpec((rb, d), lambda i: (i, 0)),
            pl.BlockSpec((d, d), lambda i: (0, 0)),
            pl.BlockSpec((1, d), lambda i: (0, 0)),
        ],
        out_specs=pl.BlockSpec((rb, d), lambda i: (i, 0)),
        out_shape=jax.ShapeDtypeStruct((n, d), jnp.float32),
    )(x, W.T, b.reshape(1, d))

    assert n < (1 << 14)
    k = 64
    grain = NW * k * NR
    e_pad = ((e + grain - 1) // grain) * grain
    chunks = e_pad // (NW * k)
    acc_rows = ((n + 1 + NS * 8 - 1) // (NS * 8)) * (NS * 8)
    src = adj[0].astype(jnp.int32)
    dst = adj[1].astype(jnp.int32)
    if e_pad != e:
        fill = jnp.arange(e_pad - e, dtype=jnp.int32)
        src = jnp.concatenate([src, fill % n])
        dst = jnp.concatenate([dst, n + fill % (acc_rows - n)])
    packed = (dst * (1 << 14) + src).reshape(NW, chunks * k)
    zeros = jnp.zeros((acc_rows, d), jnp.float32)
    partials = _sc_aggregate(x_tan, packed, zeros,
                             n=n, d=d, acc_rows=acc_rows,
                             chunks=chunks, k=k)

    out = pl.pallas_call(
        _tail_body,
        grid=(grid_h,),
        in_specs=[
            pl.BlockSpec((rb, d), lambda i: (i, 0)),
            pl.BlockSpec((rb, d), lambda i: (i, 0)),
        ],
        out_specs=pl.BlockSpec((rb, d), lambda i: (i, 0)),
        out_shape=jax.ShapeDtypeStruct((n, d), jnp.float32),
    )(partials[0], partials[1])
    return out


# speedup vs baseline: 2.7093x; 1.0274x over previous
import functools

import jax
import jax.numpy as jnp
from jax import lax
from jax.experimental import pallas as pl
from jax.experimental.pallas import tpu as pltpu
from jax.experimental.pallas import tpu_sc as plsc

EPS = 1e-15
MAX_TANH = 15.0

NC = 2
NS = 16
NW = NC * NS


def _artanh(x):
    x = jnp.clip(x, -1 + 1e-7, 1 - 1e-7)
    return 0.5 * (jnp.log1p(x) - jnp.log1p(-x))


def _safe_norm(x):
    return jnp.sqrt(jnp.clip(jnp.sum(x * x, axis=-1, keepdims=True), EPS, None))


def _proj(x):
    norm = _safe_norm(x)
    maxnorm = 1.0 - 1e-3
    return jnp.where(norm > maxnorm, x / norm * maxnorm, x)


def _expmap0(u):
    u_norm = _safe_norm(u)
    return jnp.tanh(jnp.clip(u_norm, -MAX_TANH, MAX_TANH)) * u / u_norm


def _logmap0(x):
    x_norm = _safe_norm(x)
    return _artanh(x_norm) * x / x_norm


MXN = 1.0 - 1e-3
SQEPS = 1e-15 ** 0.5


def _head_body(x_ref, wt_ref, b_ref, o_ref):
    x = x_ref[...]
    r = jnp.sqrt(jnp.clip(jnp.sum(x * x, axis=-1, keepdims=True), EPS, None))
    xh = jnp.minimum(r, MXN)
    p = xh / r
    mxr = jnp.dot(x, wt_ref[...], preferred_element_type=jnp.float32)
    q = jnp.sqrt(jnp.clip(jnp.sum(mxr * mxr, axis=-1, keepdims=True),
                          EPS, None)) * p
    mq = jnp.maximum(q, SQEPS)
    arg = jnp.clip(mq / xh * _artanh(xh), -MAX_TANH, MAX_TANH)
    s_mv = p * jnp.tanh(arg) / mq
    tmv = q / p * s_mv
    mvn = jnp.maximum(tmv, SQEPS)
    s_mv = s_mv * jnp.minimum(1.0, MXN / mvn)
    x2 = jnp.minimum(tmv, MXN) ** 2
    bias = _proj(_expmap0(b_ref[...]))
    y2 = jnp.sum(bias * bias, axis=-1, keepdims=True)
    xy = s_mv * jnp.sum(mxr * bias, axis=-1, keepdims=True)
    aa = 1 + 2 * xy + y2
    bb = 1 - x2
    den = jnp.clip(1 + 2 * xy + x2 * y2, EPS, None)
    nn2 = jnp.clip(aa * aa * x2 + 2 * aa * bb * xy + bb * bb * y2, 0.0, None)
    hn = jnp.maximum(jnp.sqrt(jnp.clip(nn2, EPS, None)) / den, SQEPS)
    prh = jnp.minimum(1.0, MXN / hn)
    hfn = jnp.maximum(hn * prh, SQEPS)
    st = (prh / den) * (_artanh(hfn) / hfn)
    o_ref[...] = (aa * s_mv * st) * mxr + (bb * st) * bias


def _tail_body(a_ref, b_ref, o_ref, s_ref):
    agg = a_ref[...] + b_ref[...]
    s_ref[...] = agg
    r02 = jnp.sum(agg * agg, axis=-1, keepdims=True)
    r0 = jnp.sqrt(jnp.clip(r02, EPS, None))
    te = jnp.tanh(jnp.minimum(r0, MAX_TANH))
    pr = jnp.minimum(1.0, MXN / jnp.maximum(te, SQEPS))
    hn1 = jnp.maximum(te * pr, SQEPS)
    s_a = (te / r0) * pr * (_artanh(hn1) / hn1)
    u = jax.nn.relu(s_ref[...] * s_a)
    s_ref[...] = u
    ru2 = jnp.sum(u * u, axis=-1, keepdims=True)
    r1 = jnp.sqrt(jnp.clip(ru2, EPS, None))
    tu = jnp.tanh(jnp.minimum(r1, MAX_TANH))
    pr2 = jnp.minimum(1.0, MXN / jnp.maximum(tu, SQEPS))
    hn2 = jnp.maximum(tu * pr2, SQEPS)
    s_b = (tu / r1) * pr2 * (_artanh(hn2) / hn2)
    o_ref[...] = s_ref[...] * s_b


NR = 4


def _sc_aggregate(x_tan, packed, *, n, d, acc_rows, chunks, k):
    rpt = acc_rows // NS
    mesh = plsc.VectorSubcoreMesh(core_axis_name="c", subcore_axis_name="s")

    @functools.partial(
        pl.kernel,
        mesh=mesh,
        out_type=jax.ShapeDtypeStruct((NC, acc_rows, d), jnp.float32),
        scratch_types=[
            pltpu.VMEM_SHARED((acc_rows, d), jnp.float32),
            pltpu.VMEM((chunks * k,), jnp.int32),
        ]
        + [pltpu.VMEM((2, k), jnp.int32) for _ in range(NR)]
        + [pltpu.VMEM((k, d), jnp.float32) for _ in range(NR)]
        + [pltpu.SemaphoreType.DMA for _ in range(2 * NR)],
    )
    def agg_kernel(xtan_hbm, packed_hbm, out_hbm,
                   acc, slab, *rest):
        idxb = rest[:NR]
        rows = rest[NR:2 * NR]
        gsems = rest[2 * NR:3 * NR]
        ssems = rest[3 * NR:]
        c = lax.axis_index("c")
        s = lax.axis_index("s")
        wid = s * NC + c

        def unpack(g, j):
            for i in range(k // 16):
                v = slab[pl.ds(g * k + i * 16, 16)]
                idxb[j].at[0][pl.ds(i * 16, 16)] = v & 0x3FFF
                idxb[j].at[1][pl.ds(i * 16, 16)] = v >> 14

        def start_gather(g, j):
            pltpu.async_copy(xtan_hbm.at[idxb[j].at[0]], rows[j], gsems[j])

        def wait_gather(j):
            pltpu.make_async_copy(
                xtan_hbm.at[idxb[j].at[0]], rows[j], gsems[j]).wait()

        def start_scatter(j):
            pltpu.async_copy(rows[j], acc.at[idxb[j].at[1]], ssems[j],
                             add=True)

        def wait_scatter(j):
            pltpu.make_async_copy(
                rows[j], acc.at[idxb[j].at[1]], ssems[j]).wait()

        pltpu.sync_copy(packed_hbm.at[wid], slab)
        zv = jnp.zeros((16,), jnp.float32)
        for i in range(k * d // 16):
            rows[0].at[i // (d // 16)][pl.ds((i % (d // 16)) * 16, 16)] = zv
        z0 = pl.multiple_of(s * rpt, 8)
        nfull, rem = rpt // k, rpt % k
        for i in range(nfull):
            pltpu.async_copy(rows[0], acc.at[pl.ds(z0 + i * k, k)], ssems[0])
        if rem:
            pltpu.async_copy(rows[0].at[pl.ds(0, rem)],
                             acc.at[pl.ds(z0 + nfull * k, rem)], ssems[0])
        for i in range(nfull):
            pltpu.make_async_copy(
                rows[0], acc.at[pl.ds(z0 + i * k, k)], ssems[0]).wait()
        if rem:
            pltpu.make_async_copy(
                rows[0].at[pl.ds(0, rem)],
                acc.at[pl.ds(z0 + nfull * k, rem)], ssems[0]).wait()
        for j in range(NR):
            unpack(j, j)
            start_gather(j, j)
        plsc.subcore_barrier()

        def outer(t, carry):
            for j in range(NR):
                wait_gather(j)
                start_scatter(j)
            for j in range(NR):
                gf = t * NR + j + NR

                @pl.when(gf < chunks)
                def _refill(j=j, gf=gf):
                    wait_scatter(j)
                    unpack(gf, j)
                    start_gather(gf, j)
            return carry

        lax.fori_loop(0, chunks // NR, outer, 0)
        for j in range(NR):
            wait_scatter(j)
        plsc.subcore_barrier()
        pltpu.sync_copy(acc.at[pl.ds(z0, rpt)],
                        out_hbm.at[c, pl.ds(z0, rpt)])

    return agg_kernel(x_tan, packed)


def kernel(x, adj, W, b):
    n, d = x.shape
    e = adj.shape[1]
    assert n % NS == 0 and d % 128 == 0

    rb = 400
    grid_h = n // rb
    x_tan = pl.pallas_call(
        _head_body,
        grid=(grid_h,),
        in_specs=[
            pl.BlockSpec((rb, d), lambda i: (i, 0)),
            pl.BlockSpec((d, d), lambda i: (0, 0)),
            pl.BlockSpec((1, d), lambda i: (0, 0)),
        ],
        out_specs=pl.BlockSpec((rb, d), lambda i: (i, 0)),
        out_shape=jax.ShapeDtypeStruct((n, d), jnp.float32),
    )(x, W.T, b.reshape(1, d))

    assert n < (1 << 14)
    k = 64
    grain = NW * k * NR
    e_pad = ((e + grain - 1) // grain) * grain
    chunks = e_pad // (NW * k)
    acc_rows = ((n + 1 + NS * 8 - 1) // (NS * 8)) * (NS * 8)
    src = adj[0].astype(jnp.int32)
    dst = adj[1].astype(jnp.int32)
    if e_pad != e:
        fill = jnp.arange(e_pad - e, dtype=jnp.int32)
        src = jnp.concatenate([src, fill % n])
        dst = jnp.concatenate([dst, n + fill % (acc_rows - n)])
    packed = (dst * (1 << 14) + src).reshape(NW, chunks * k)
    partials = _sc_aggregate(x_tan, packed,
                             n=n, d=d, acc_rows=acc_rows,
                             chunks=chunks, k=k)

    out = pl.pallas_call(
        _tail_body,
        grid=(grid_h,),
        in_specs=[
            pl.BlockSpec((rb, d), lambda i: (i, 0)),
            pl.BlockSpec((rb, d), lambda i: (i, 0)),
        ],
        out_specs=pl.BlockSpec((rb, d), lambda i: (i, 0)),
        out_shape=jax.ShapeDtypeStruct((n, d), jnp.float32),
        scratch_shapes=[pltpu.VMEM((rb, d), jnp.float32)],
    )(partials[0], partials[1])
    return out
